# Initial kernel scaffold; baseline (speedup 1.0000x reference)
import jax
import jax.numpy as jnp
from jax.experimental import pallas as pl


def kernel(x_0, x_1, adj_0_0, adj_0_1, adj_1_1, inv_0_0, inv_0_1, inv_1_1, Wm00, bm00, We00, be00, Wm01, bm01, We01, be01, Wm11, bm11, We11, be11, Wu0, bu0, Wu1, bu1):
    raise NotImplementedError("write your pallas kernel here")


# speedup vs baseline: 1.6254x; 1.6254x over previous
import functools

import jax
import jax.numpy as jnp
from jax import lax
from jax.experimental import pallas as pl
from jax.experimental.pallas import tpu as pltpu
from jax.experimental.pallas import tpu_sc as plsc

N0, N1, H = 10000, 20000, 128
E00, E01, E11 = 320000, 40000, 320000
CE = 80
ACC_ROWS = 10240
TRASH = 10200
HALF = 10000


def _xform(x, Ws, BR):
    N = x.shape[0]
    nw = len(Ws)

    def body(x_ref, *refs):
        xv = x_ref[...]
        for wr, orf in zip(refs[:nw], refs[nw:]):
            orf[...] = jnp.dot(xv, wr[...], preferred_element_type=jnp.float32)

    return pl.pallas_call(
        body,
        grid=(N // BR,),
        in_specs=[pl.BlockSpec((BR, H), lambda i: (i, 0))]
        + [pl.BlockSpec((H, H), lambda i: (0, 0))] * nw,
        out_specs=[pl.BlockSpec((BR, H), lambda i: (i, 0))] * nw,
        out_shape=[jax.ShapeDtypeStruct((N, H), jnp.float32)] * nw,
    )(x, *Ws)


def _cmat(inv, Wi, bm, BR):
    E, K = inv.shape

    def body(i_ref, w_ref, b_ref, o_ref):
        o_ref[...] = (
            jnp.dot(i_ref[...], w_ref[...], preferred_element_type=jnp.float32)
            + b_ref[...]
        )

    return pl.pallas_call(
        body,
        grid=(E // BR,),
        in_specs=[
            pl.BlockSpec((BR, K), lambda i: (i, 0)),
            pl.BlockSpec((K, H), lambda i: (0, 0)),
            pl.BlockSpec((1, H), lambda i: (0, 0)),
        ],
        out_specs=pl.BlockSpec((BR, H), lambda i: (i, 0)),
        out_shape=jax.ShapeDtypeStruct((E, H), jnp.float32),
    )(inv, Wi, bm[None, :])


def _update0(x0, m00p, Wu0, bu0, BR):
    def body(x_ref, m_ref, w_ref, b_ref, o_ref):
        xv = x_ref[...]
        mv = m_ref[0] + m_ref[1]
        o_ref[...] = (
            xv
            + jnp.dot(xv, w_ref[:H, :], preferred_element_type=jnp.float32)
            + jnp.dot(mv, w_ref[H:, :], preferred_element_type=jnp.float32)
            + b_ref[...]
        )

    return pl.pallas_call(
        body,
        grid=(N0 // BR,),
        in_specs=[
            pl.BlockSpec((BR, H), lambda i: (i, 0)),
            pl.BlockSpec((2, BR, H), lambda i: (0, i, 0)),
            pl.BlockSpec((2 * H, H), lambda i: (0, 0)),
            pl.BlockSpec((1, H), lambda i: (0, 0)),
        ],
        out_specs=pl.BlockSpec((BR, H), lambda i: (i, 0)),
        out_shape=jax.ShapeDtypeStruct((N0, H), jnp.float32),
    )(x0, m00p, Wu0, bu0[None, :])


def _update1(x1, m01, m11, Wu1, bu1, BR):
    def body(x_ref, ma_ref, mb_ref, w_ref, b_ref, o_ref):
        xv = x_ref[...]
        o_ref[...] = (
            xv
            + jnp.dot(xv, w_ref[:H, :], preferred_element_type=jnp.float32)
            + jnp.dot(ma_ref[...], w_ref[H : 2 * H, :],
                      preferred_element_type=jnp.float32)
            + jnp.dot(mb_ref[...], w_ref[2 * H :, :],
                      preferred_element_type=jnp.float32)
            + b_ref[...]
        )

    return pl.pallas_call(
        body,
        grid=(N1 // BR,),
        in_specs=[
            pl.BlockSpec((BR, H), lambda i: (i, 0)),
            pl.BlockSpec((BR, H), lambda i: (i, 0)),
            pl.BlockSpec((BR, H), lambda i: (i, 0)),
            pl.BlockSpec((3 * H, H), lambda i: (0, 0)),
            pl.BlockSpec((1, H), lambda i: (0, 0)),
        ],
        out_specs=pl.BlockSpec((BR, H), lambda i: (i, 0)),
        out_shape=jax.ShapeDtypeStruct((N1, H), jnp.float32),
    )(x1, m01, m11, Wu1, bu1[None, :])


def _sc_passes(
    s00, r00, c00, i00s, i00r,
    s01, r01, c01, i01s, i01r,
    s11, r11, c11, i11s, i11r,
    gparams,
    m00p, m01, m11,
    acc, bufc, bufs, bufr, outb, idx0v, idx1v, sidxv, gpv,
    sem_s, sem_r,
):
    cid = lax.axis_index("c")
    sid = lax.axis_index("s")
    zv = jnp.zeros((16,), jnp.float32)
    lane = lax.iota(jnp.int32, 16)
    bfly = [lane ^ (1 << b) for b in range(4)]

    gdn = lax.GatherDimensionNumbers(
        offset_dims=(), collapsed_slice_dims=(0,), start_index_map=(0,))

    def lanesum(v):
        for p in bfly:
            v = v + lax.gather(v, p[:, None], gdn, (1,),
                               mode=lax.GatherScatterMode.PROMISE_IN_BOUNDS)
        return v

    pltpu.sync_copy(gparams, gpv)

    def zero_acc():
        def _zrow(r, carry):
            for k in range(8):
                outb[r, pl.ds(16 * k, 16)] = zv
            return carry

        lax.fori_loop(0, CE, _zrow, 0)
        r0 = sid * (ACC_ROWS // 16)
        for q in range(ACC_ROWS // 16 // CE):
            pltpu.sync_copy(outb, acc.at[pl.ds(r0 + q * CE, CE)])

    def do_chunk(base, Sh, Rh, Ch, i0h, i1h, split, woff, boff):
        pltpu.sync_copy(i0h.at[pl.ds(base, CE)], idx0v)
        pltpu.sync_copy(i1h.at[pl.ds(base, CE)], idx1v)
        pltpu.sync_copy(Ch.at[pl.ds(base, CE)], bufc)
        cs = pltpu.async_copy(Sh.at[idx0v], bufs, sem_s)
        cr = pltpu.async_copy(Rh.at[idx1v], bufr, sem_r)
        for t in range(CE // 16):
            iv = idx1v[pl.ds(16 * t, 16)]
            if split:
                rr = iv - cid * HALF
                msk = (rr >= 0) & (rr < HALF)
                rr = jnp.where(msk, rr, TRASH)
            else:
                rr = iv
            sidxv[pl.ds(16 * t, 16)] = rr
        cs.wait()
        cr.wait()

        def ebody(e, carry):
            tacc = zv
            ms = []
            for k in range(8):
                p = (
                    bufc[e, pl.ds(16 * k, 16)]
                    + bufs[e, pl.ds(16 * k, 16)]
                    + bufr[e, pl.ds(16 * k, 16)]
                )
                mk = p / (1.0 + jnp.exp(-p))
                ms.append(mk)
                tacc = tacc + mk * gpv[pl.ds(woff + 16 * k, 16)]
            tv = lanesum(tacc) + gpv[pl.ds(boff, 16)]
            wv = 1.0 / (1.0 + jnp.exp(-tv))
            for k in range(8):
                outb[e, pl.ds(16 * k, 16)] = ms[k] * wv
            return carry

        lax.fori_loop(0, CE, ebody, 0)
        pltpu.sync_copy(outb, acc.at[sidxv], add=True)

    def flush(dst_fn):
        for j in range(8):
            gid = sid + j * 16

            @pl.when(gid < 125)
            def _():
                row = gid * CE
                pltpu.sync_copy(acc.at[pl.ds(row, CE)], outb)
                pltpu.sync_copy(outb, dst_fn(row))

    zero_acc()
    plsc.subcore_barrier()

    def j00(j, carry):
        gid = sid + j * 16
        do_chunk(cid * (E00 // 2) + gid * CE, s00, r00, c00, i00s, i00r,
                 False, 0, 384)
        return carry

    lax.fori_loop(0, E00 // 2 // CE // 16, j00, 0)
    plsc.subcore_barrier()
    flush(lambda row: m00p.at[cid, pl.ds(row, CE)])
    plsc.subcore_barrier()

    zero_acc()
    plsc.subcore_barrier()
    nch01 = E01 // CE

    def j01(j, carry):
        gid = sid + j * 16

        @pl.when(gid < nch01)
        def _():
            do_chunk(gid * CE, s01, r01, c01, i01s, i01r, True, 128, 400)

        return carry

    lax.fori_loop(0, (nch01 + 15) // 16, j01, 0)
    plsc.subcore_barrier()
    flush(lambda row: m01.at[pl.ds(cid * HALF + row, CE)])
    plsc.subcore_barrier()

    zero_acc()
    plsc.subcore_barrier()

    def j11(j, carry):
        gid = sid + j * 16
        do_chunk(gid * CE, s11, r11, c11, i11s, i11r, True, 256, 416)
        return carry

    lax.fori_loop(0, E11 // CE // 16, j11, 0)
    plsc.subcore_barrier()
    flush(lambda row: m11.at[pl.ds(cid * HALF + row, CE)])


def _sc_messages(s00, r00, c00, i00s, i00r, s01, r01, c01, i01s, i01r,
                 s11, r11, c11, i11s, i11r, gparams):
    mesh = plsc.VectorSubcoreMesh(
        core_axis_name="c", subcore_axis_name="s", num_cores=2, num_subcores=16
    )
    f = pl.kernel(
        _sc_passes,
        out_type=[
            jax.ShapeDtypeStruct((2, N0, H), jnp.float32),
            jax.ShapeDtypeStruct((N1, H), jnp.float32),
            jax.ShapeDtypeStruct((N1, H), jnp.float32),
        ],
        mesh=mesh,
        scratch_types=[
            pltpu.VMEM_SHARED((ACC_ROWS, H), jnp.float32),
            pltpu.VMEM((CE, H), jnp.float32),
            pltpu.VMEM((CE, H), jnp.float32),
            pltpu.VMEM((CE, H), jnp.float32),
            pltpu.VMEM((CE, H), jnp.float32),
            pltpu.VMEM((CE,), jnp.int32),
            pltpu.VMEM((CE,), jnp.int32),
            pltpu.VMEM((CE,), jnp.int32),
            pltpu.VMEM((448,), jnp.float32),
            pltpu.SemaphoreType.DMA,
            pltpu.SemaphoreType.DMA,
        ],
    )
    return f(s00, r00, c00, i00s, i00r, s01, r01, c01, i01s, i01r,
             s11, r11, c11, i11s, i11r, gparams)


def kernel(x_0, x_1, adj_0_0, adj_0_1, adj_1_1, inv_0_0, inv_0_1, inv_1_1,
           Wm00, bm00, We00, be00, Wm01, bm01, We01, be01,
           Wm11, bm11, We11, be11, Wu0, bu0, Wu1, bu1):
    i00s = adj_0_0[0].astype(jnp.int32)
    i00r = adj_0_0[1].astype(jnp.int32)
    i01s = adj_0_1[0].astype(jnp.int32)
    i01r = adj_0_1[1].astype(jnp.int32)
    i11s = adj_1_1[0].astype(jnp.int32)
    i11r = adj_1_1[1].astype(jnp.int32)

    s00, r00, s01 = _xform(x_0, [Wm00[:H], Wm00[H:2 * H], Wm01[:H]], 1000)
    r01, s11, r11 = _xform(x_1, [Wm01[H:2 * H], Wm11[:H], Wm11[H:2 * H]], 1000)
    c00 = _cmat(inv_0_0, Wm00[2 * H:], bm00, 4000)
    c01 = _cmat(inv_0_1, Wm01[2 * H:], bm01, 4000)
    c11 = _cmat(inv_1_1, Wm11[2 * H:], bm11, 4000)

    gparams = jnp.concatenate([
        We00[:, 0], We01[:, 0], We11[:, 0],
        jnp.full((16,), be00[0], jnp.float32),
        jnp.full((16,), be01[0], jnp.float32),
        jnp.full((16,), be11[0], jnp.float32),
        jnp.zeros((16,), jnp.float32),
    ])

    m00p, m01, m11 = _sc_messages(
        s00, r00, c00, i00s, i00r, s01, r01, c01, i01s, i01r,
        s11, r11, c11, i11s, i11r, gparams)

    out0 = _update0(x_0, m00p, Wu0, bu0, 1000)
    out1 = _update1(x_1, m01, m11, Wu1, bu1, 1000)
    return (out0, out1)


# speedup vs baseline: 1.8106x; 1.1139x over previous
import functools

import jax
import jax.numpy as jnp
from jax import lax
from jax.experimental import pallas as pl
from jax.experimental.pallas import tpu as pltpu
from jax.experimental.pallas import tpu_sc as plsc

N0, N1, H = 10000, 20000, 128
E00, E01, E11 = 320000, 40000, 320000
CE = 64
ACC_ROWS = 10112
TRASH = 10050
HALF = 10000


def _xform(x, Ws, BR):
    N = x.shape[0]
    nw = len(Ws)

    def body(x_ref, *refs):
        xv = x_ref[...]
        for wr, orf in zip(refs[:nw], refs[nw:]):
            orf[...] = jnp.dot(xv, wr[...], preferred_element_type=jnp.float32)

    return pl.pallas_call(
        body,
        grid=(N // BR,),
        in_specs=[pl.BlockSpec((BR, H), lambda i: (i, 0))]
        + [pl.BlockSpec((H, H), lambda i: (0, 0))] * nw,
        out_specs=[pl.BlockSpec((BR, H), lambda i: (i, 0))] * nw,
        out_shape=[jax.ShapeDtypeStruct((N, H), jnp.float32)] * nw,
    )(x, *Ws)


def _cmat(inv, Wi, bm, BR):
    E, K = inv.shape

    def body(i_ref, w_ref, b_ref, o_ref):
        o_ref[...] = (
            jnp.dot(i_ref[...], w_ref[...], preferred_element_type=jnp.float32)
            + b_ref[...]
        )

    return pl.pallas_call(
        body,
        grid=(E // BR,),
        in_specs=[
            pl.BlockSpec((BR, K), lambda i: (i, 0)),
            pl.BlockSpec((K, H), lambda i: (0, 0)),
            pl.BlockSpec((1, H), lambda i: (0, 0)),
        ],
        out_specs=pl.BlockSpec((BR, H), lambda i: (i, 0)),
        out_shape=jax.ShapeDtypeStruct((E, H), jnp.float32),
    )(inv, Wi, bm[None, :])


def _update0(x0, m00p, Wu0, bu0, BR):
    def body(x_ref, m_ref, w_ref, b_ref, o_ref):
        xv = x_ref[...]
        mv = m_ref[0] + m_ref[1]
        o_ref[...] = (
            xv
            + jnp.dot(xv, w_ref[:H, :], preferred_element_type=jnp.float32)
            + jnp.dot(mv, w_ref[H:, :], preferred_element_type=jnp.float32)
            + b_ref[...]
        )

    return pl.pallas_call(
        body,
        grid=(N0 // BR,),
        in_specs=[
            pl.BlockSpec((BR, H), lambda i: (i, 0)),
            pl.BlockSpec((2, BR, H), lambda i: (0, i, 0)),
            pl.BlockSpec((2 * H, H), lambda i: (0, 0)),
            pl.BlockSpec((1, H), lambda i: (0, 0)),
        ],
        out_specs=pl.BlockSpec((BR, H), lambda i: (i, 0)),
        out_shape=jax.ShapeDtypeStruct((N0, H), jnp.float32),
    )(x0, m00p, Wu0, bu0[None, :])


def _update1(x1, m01, m11, Wu1, bu1, BR):
    def body(x_ref, ma_ref, mb_ref, w_ref, b_ref, o_ref):
        xv = x_ref[...]
        o_ref[...] = (
            xv
            + jnp.dot(xv, w_ref[:H, :], preferred_element_type=jnp.float32)
            + jnp.dot(ma_ref[...], w_ref[H : 2 * H, :],
                      preferred_element_type=jnp.float32)
            + jnp.dot(mb_ref[...], w_ref[2 * H :, :],
                      preferred_element_type=jnp.float32)
            + b_ref[...]
        )

    return pl.pallas_call(
        body,
        grid=(N1 // BR,),
        in_specs=[
            pl.BlockSpec((BR, H), lambda i: (i, 0)),
            pl.BlockSpec((BR, H), lambda i: (i, 0)),
            pl.BlockSpec((BR, H), lambda i: (i, 0)),
            pl.BlockSpec((3 * H, H), lambda i: (0, 0)),
            pl.BlockSpec((1, H), lambda i: (0, 0)),
        ],
        out_specs=pl.BlockSpec((BR, H), lambda i: (i, 0)),
        out_shape=jax.ShapeDtypeStruct((N1, H), jnp.float32),
    )(x1, m01, m11, Wu1, bu1[None, :])


def _sc_passes(
    s00, r00, c00, i00s, i00r,
    s01, r01, c01, i01s, i01r,
    s11, r11, c11, i11s, i11r,
    gparams,
    m00p, m01, m11,
    acc, bufc0, bufc1, bufs0, bufs1, bufr0, bufr1,
    idx0a, idx0b, idx1a, idx1b, gpv,
    ia0, ia1, ib0, ib1, ic0, ic1, gsem0, gsem1,
):
    cid = lax.axis_index("c")
    sid = lax.axis_index("s")
    zv = jnp.zeros((16,), jnp.float32)
    lane = lax.iota(jnp.int32, 16)
    bfly = [lane ^ (1 << b) for b in range(4)]
    bufc = (bufc0, bufc1)
    bufs = (bufs0, bufs1)
    bufr = (bufr0, bufr1)
    idx0v = (idx0a, idx0b)
    idx1v = (idx1a, idx1b)
    isem0v = (ia0, ia1)
    isem1v = (ib0, ib1)
    isemc = (ic0, ic1)
    gsem = (gsem0, gsem1)

    gdn = lax.GatherDimensionNumbers(
        offset_dims=(), collapsed_slice_dims=(0,), start_index_map=(0,))

    def lanesum(v):
        for p in bfly:
            v = v + lax.gather(v, p[:, None], gdn, (1,),
                               mode=lax.GatherScatterMode.PROMISE_IN_BOUNDS)
        return v

    pltpu.sync_copy(gparams, gpv)

    def zero_acc():
        def _zrow(r, carry):
            for k in range(8):
                bufc0[r, pl.ds(16 * k, 16)] = zv
            return carry

        lax.fori_loop(0, CE, _zrow, 0)
        for q in range(10):
            z = sid + 16 * q

            @pl.when(z < ACC_ROWS // CE)
            def _():
                pltpu.sync_copy(bufc0, acc.at[pl.ds(z * CE, CE)])

    def flush(dst64, dst16):
        for q in range(10):
            z = sid + 16 * q

            @pl.when(z < HALF // CE)
            def _():
                row = z * CE
                pltpu.sync_copy(acc.at[pl.ds(row, CE)], bufc0)
                pltpu.sync_copy(bufc0, dst64(row))

        @pl.when(sid == 0)
        def _():
            row = (HALF // CE) * CE
            pltpu.sync_copy(acc.at[pl.ds(row, 16)], bufc0.at[pl.ds(0, 16)])
            pltpu.sync_copy(bufc0.at[pl.ds(0, 16)], dst16(row))

    def run_pass(Sh, Rh, Ch, i0h, i1h, nch, base_off, split, woff, boff):
        def issue_inputs(j, slot):
            gid = sid + 16 * j

            @pl.when(gid < nch)
            def _():
                b = base_off + gid * CE
                pltpu.async_copy(i0h.at[pl.ds(b, CE)], idx0v[slot], isem0v[slot])
                pltpu.async_copy(i1h.at[pl.ds(b, CE)], idx1v[slot], isem1v[slot])
                pltpu.async_copy(Ch.at[pl.ds(b, CE)], bufc[slot], isemc[slot])

        def wait_inputs(slot):
            pltpu.make_async_copy(i0h.at[pl.ds(0, CE)], idx0v[slot], isem0v[slot]).wait()
            pltpu.make_async_copy(i1h.at[pl.ds(0, CE)], idx1v[slot], isem1v[slot]).wait()
            pltpu.make_async_copy(Ch.at[pl.ds(0, CE)], bufc[slot], isemc[slot]).wait()

        def issue_gathers(slot):
            pltpu.async_copy(Sh.at[idx0v[slot]], bufs[slot], gsem[slot])
            pltpu.async_copy(Rh.at[idx1v[slot]], bufr[slot], gsem[slot])

        def wait_gathers(slot):
            pltpu.make_async_copy(Sh.at[idx0v[slot]], bufs[slot], gsem[slot]).wait()
            pltpu.make_async_copy(Rh.at[idx1v[slot]], bufr[slot], gsem[slot]).wait()

        def compute_scatter(slot):
            if split:
                for t in range(CE // 16):
                    iv = idx1v[slot][pl.ds(16 * t, 16)]
                    rr = iv - cid * HALF
                    msk = (rr >= 0) & (rr < HALF)
                    idx1v[slot][pl.ds(16 * t, 16)] = jnp.where(msk, rr, TRASH)
            wes = tuple(gpv[pl.ds(woff + 16 * k, 16)] for k in range(8))
            bev = gpv[pl.ds(boff, 16)]
            cb, sb, rb = bufc[slot], bufs[slot], bufr[slot]

            @functools.partial(lax.fori_loop, 0, CE, init_val=(wes, bev))
            def _eloop(e, cw):
                ws, be = cw
                tacc = zv
                ms = []
                for k in range(8):
                    sl = pl.ds(16 * k, 16)
                    p = cb[e, sl] + sb[e, sl] + rb[e, sl]
                    mk = p / (1.0 + jnp.exp(-p))
                    ms.append(mk)
                    tacc = tacc + mk * ws[k]
                wv = 1.0 / (1.0 + jnp.exp(-(lanesum(tacc) + be)))
                for k in range(8):
                    cb[e, pl.ds(16 * k, 16)] = ms[k] * wv
                return cw

            pltpu.sync_copy(cb, acc.at[idx1v[slot]], add=True)

        issue_inputs(0, 0)
        issue_inputs(1, 1)
        wait_inputs(0)
        issue_gathers(0)

        jmax = (nch + 15) // 16

        def body(j2, carry):
            for sub in (0, 1):
                j = 2 * j2 + sub
                s, o = sub, 1 - sub
                g0 = sid + 16 * j

                @pl.when(sid + 16 * (j + 1) < nch)
                def _():
                    wait_inputs(o)
                    issue_gathers(o)

                @pl.when(g0 < nch)
                def _():
                    wait_gathers(s)
                    compute_scatter(s)

                issue_inputs(j + 2, s)
            return carry

        lax.fori_loop(0, (jmax + 1) // 2, body, 0)

    zero_acc()
    plsc.subcore_barrier()
    run_pass(s00, r00, c00, i00s, i00r, E00 // 2 // CE, cid * (E00 // 2),
             False, 0, 384)
    plsc.subcore_barrier()
    flush(lambda row: m00p.at[cid, pl.ds(row, CE)],
          lambda row: m00p.at[cid, pl.ds(row, 16)])
    plsc.subcore_barrier()

    zero_acc()
    plsc.subcore_barrier()
    run_pass(s01, r01, c01, i01s, i01r, E01 // CE, 0, True, 128, 400)
    plsc.subcore_barrier()
    flush(lambda row: m01.at[pl.ds(cid * HALF + row, CE)],
          lambda row: m01.at[pl.ds(cid * HALF + row, 16)])
    plsc.subcore_barrier()

    zero_acc()
    plsc.subcore_barrier()
    run_pass(s11, r11, c11, i11s, i11r, E11 // CE, 0, True, 256, 416)
    plsc.subcore_barrier()
    flush(lambda row: m11.at[pl.ds(cid * HALF + row, CE)],
          lambda row: m11.at[pl.ds(cid * HALF + row, 16)])


def _sc_messages(s00, r00, c00, i00s, i00r, s01, r01, c01, i01s, i01r,
                 s11, r11, c11, i11s, i11r, gparams):
    mesh = plsc.VectorSubcoreMesh(
        core_axis_name="c", subcore_axis_name="s", num_cores=2, num_subcores=16
    )
    f = pl.kernel(
        _sc_passes,
        out_type=[
            jax.ShapeDtypeStruct((2, N0, H), jnp.float32),
            jax.ShapeDtypeStruct((N1, H), jnp.float32),
            jax.ShapeDtypeStruct((N1, H), jnp.float32),
        ],
        mesh=mesh,
        scratch_types=[
            pltpu.VMEM_SHARED((ACC_ROWS, H), jnp.float32),
            pltpu.VMEM((CE, H), jnp.float32),
            pltpu.VMEM((CE, H), jnp.float32),
            pltpu.VMEM((CE, H), jnp.float32),
            pltpu.VMEM((CE, H), jnp.float32),
            pltpu.VMEM((CE, H), jnp.float32),
            pltpu.VMEM((CE, H), jnp.float32),
            pltpu.VMEM((CE,), jnp.int32),
            pltpu.VMEM((CE,), jnp.int32),
            pltpu.VMEM((CE,), jnp.int32),
            pltpu.VMEM((CE,), jnp.int32),
            pltpu.VMEM((448,), jnp.float32),
            pltpu.SemaphoreType.DMA,
            pltpu.SemaphoreType.DMA,
            pltpu.SemaphoreType.DMA,
            pltpu.SemaphoreType.DMA,
            pltpu.SemaphoreType.DMA,
            pltpu.SemaphoreType.DMA,
            pltpu.SemaphoreType.DMA,
            pltpu.SemaphoreType.DMA,
        ],
    )
    return f(s00, r00, c00, i00s, i00r, s01, r01, c01, i01s, i01r,
             s11, r11, c11, i11s, i11r, gparams)


def kernel(x_0, x_1, adj_0_0, adj_0_1, adj_1_1, inv_0_0, inv_0_1, inv_1_1,
           Wm00, bm00, We00, be00, Wm01, bm01, We01, be01,
           Wm11, bm11, We11, be11, Wu0, bu0, Wu1, bu1):
    i00s = adj_0_0[0].astype(jnp.int32)
    i00r = adj_0_0[1].astype(jnp.int32)
    i01s = adj_0_1[0].astype(jnp.int32)
    i01r = adj_0_1[1].astype(jnp.int32)
    i11s = adj_1_1[0].astype(jnp.int32)
    i11r = adj_1_1[1].astype(jnp.int32)

    s00, r00, s01 = _xform(x_0, [Wm00[:H], Wm00[H:2 * H], Wm01[:H]], 1000)
    r01, s11, r11 = _xform(x_1, [Wm01[H:2 * H], Wm11[:H], Wm11[H:2 * H]], 1000)
    c00 = _cmat(inv_0_0, Wm00[2 * H:], bm00, 4000)
    c01 = _cmat(inv_0_1, Wm01[2 * H:], bm01, 4000)
    c11 = _cmat(inv_1_1, Wm11[2 * H:], bm11, 4000)

    gparams = jnp.concatenate([
        We00[:, 0], We01[:, 0], We11[:, 0],
        jnp.full((16,), be00[0], jnp.float32),
        jnp.full((16,), be01[0], jnp.float32),
        jnp.full((16,), be11[0], jnp.float32),
        jnp.zeros((16,), jnp.float32),
    ])

    m00p, m01, m11 = _sc_messages(
        s00, r00, c00, i00s, i00r, s01, r01, c01, i01s, i01r,
        s11, r11, c11, i11s, i11r, gparams)

    out0 = _update0(x_0, m00p, Wu0, bu0, 1000)
    out1 = _update1(x_1, m01, m11, Wu1, bu1, 1000)
    return (out0, out1)
